# baseline (speedup 1.0000x reference)
import jax
import jax.numpy as jnp
import numpy as np
from jax.experimental import pallas as pl
from jax.experimental.pallas import tpu as pltpu

N_COLS = 22
N_PAIRS = N_COLS * (N_COLS - 1) // 2
EMB_NUM = 12
ROWS = N_COLS * EMB_NUM
D = 64
N_OPS = 5
TILE = 2048

_I_IDX, _J_IDX = np.triu_indices(N_COLS, k=1)
_MIR = np.zeros((ROWS, N_PAIRS), np.float32)
_MJR = np.zeros((ROWS, N_PAIRS), np.float32)
for _p in range(N_PAIRS):
    _MIR[_I_IDX[_p] * EMB_NUM:(_I_IDX[_p] + 1) * EMB_NUM, _p] = 1.0
    _MJR[_J_IDX[_p] * EMB_NUM:(_J_IDX[_p] + 1) * EMB_NUM, _p] = 1.0

_HI = jax.lax.Precision.HIGHEST
_H3 = jax.lax.Precision.HIGH

_R22 = np.zeros((N_COLS, ROWS), np.float32)
for _c in range(N_COLS):
    _R22[_c, _c * EMB_NUM:(_c + 1) * EMB_NUM] = 1.0


def _nasp_kernel(la_ref, pi_ref, pj_ref, feat_ref, mean_ref, std_ref, v_ref,
                 f0_ref, f1_ref, f2_ref, f3_ref, fa_ref, fb_ref,
                 mir_ref, mjr_ref, r22_ref, out_ref, e_ref, acc0_ref, acc1_ref,
                 sp01_ref, sbh_ref, sbl_ref, pbh_ref, pbl_ref):
    logits = [la_ref[k] for k in range(N_OPS)]
    best = logits[0]
    sel = jnp.int32(0)
    for k in range(1, N_OPS):
        is_new = logits[k] > best
        best = jnp.where(is_new, logits[k], best)
        sel = jnp.where(is_new, jnp.int32(k), sel)
    w = [(sel == k).astype(jnp.float32) for k in range(N_OPS)]
    w0, w1, w2, w3, w4 = w
    hw2 = 0.5 * w2
    hw3 = 0.5 * w3

    @pl.when(pl.program_id(0) == 0)
    def _prep():
        sp01_ref[...] = jnp.log1p(jnp.exp(std_ref[...])) * 0.01
        s_w = w0 * f0_ref[...] + hw2 * f2_ref[...] + hw3 * f3_ref[...]
        gi = s_w + w4 * fa_ref[...]
        gj = s_w + w4 * fb_ref[...]
        wcolr = (jnp.dot(mir_ref[...], gi, preferred_element_type=jnp.float32,
                         precision=_HI) +
                 jnp.dot(mjr_ref[...], gj, preferred_element_type=jnp.float32,
                         precision=_HI))
        mean = mean_ref[...]
        p0 = jnp.sum(mean * wcolr[:, :D], axis=1, keepdims=True)
        p1 = jnp.sum(mean * wcolr[:, D:], axis=1, keepdims=True)
        pbig = jnp.concatenate([p0, p1], axis=1)
        sp01 = sp01_ref[...]
        sbig = jnp.concatenate([sp01, sp01], axis=1) * wcolr
        sbh = sbig.astype(jnp.bfloat16)
        sbh_ref[...] = sbh
        sbl_ref[...] = (sbig - sbh.astype(jnp.float32)).astype(jnp.bfloat16)
        pbh = pbig.astype(jnp.bfloat16)
        pbh_ref[...] = pbh
        pbl_ref[...] = (pbig - pbh.astype(jnp.float32)).astype(jnp.bfloat16)

    iota = jax.lax.broadcasted_iota(jnp.int32, (1, ROWS), 1)
    pattern = jax.lax.rem(iota, jnp.int32(EMB_NUM)).astype(jnp.float32)
    featr = jnp.dot(feat_ref[...].astype(jnp.float32), r22_ref[...],
                    preferred_element_type=jnp.float32)
    oh = (featr == pattern).astype(jnp.float32)

    ohb = oh.astype(jnp.bfloat16)
    lin = (jnp.dot(ohb, pbh_ref[...], preferred_element_type=jnp.float32) +
           jnp.dot(ohb, pbl_ref[...], preferred_element_type=jnp.float32))
    g = (jnp.dot(ohb, sbh_ref[...], preferred_element_type=jnp.float32) +
         jnp.dot(ohb, sbl_ref[...], preferred_element_type=jnp.float32))
    v = v_ref[...]
    h = g * jnp.concatenate([v, v], axis=1)
    lin0 = lin[:, 0:1] + jnp.sum(h[:, :D], axis=1, keepdims=True)
    lin1 = lin[:, 1:2] + jnp.sum(h[:, D:], axis=1, keepdims=True)
    out_ref[...] = jnp.concatenate([lin0, lin1], axis=1)

    @pl.when(w1 + w2 + w3 > 0.5)
    def _pairwise():
        acc0_ref[...] = jnp.zeros((TILE, D), jnp.float32)
        acc1_ref[...] = jnp.zeros((TILE, D), jnp.float32)
        for c in range(N_COLS):
            ohc = oh[:, c * EMB_NUM:(c + 1) * EMB_NUM]
            mu = jnp.dot(ohc, mean_ref[c * EMB_NUM:(c + 1) * EMB_NUM, :],
                         preferred_element_type=jnp.float32, precision=_HI)
            sp = jnp.dot(ohc, sp01_ref[c * EMB_NUM:(c + 1) * EMB_NUM, :],
                         preferred_element_type=jnp.float32, precision=_HI)
            e_ref[c] = mu + sp * v
        def body(p, carry):
            i = pi_ref[p]
            j = pj_ref[p]
            ei = e_ref[i]
            ej = e_ref[j]
            m = ei * ej
            a = jnp.abs(ei - ej)
            wm = w1 * f1_ref[pl.ds(p, 1), :]
            wa = hw2 * f2_ref[pl.ds(p, 1), :] - hw3 * f3_ref[pl.ds(p, 1), :]
            acc0_ref[...] += m * wm[:, :D] + a * wa[:, :D]
            acc1_ref[...] += m * wm[:, D:] + a * wa[:, D:]
            return carry

        jax.lax.fori_loop(0, N_PAIRS, body, 0)
        r0 = jnp.sum(acc0_ref[...], axis=1, keepdims=True)
        r1 = jnp.sum(acc1_ref[...], axis=1, keepdims=True)
        out_ref[...] += jnp.concatenate([r0, r1], axis=1)


def kernel(features, label, emb_mean, emb_std, fc_small, fc_concat, log_alpha, rand_array):
    del label
    B = features.shape[1]
    featT = features.T.astype(jnp.int32)
    v = rand_array[:B * D].reshape(B, D)
    mean264 = emb_mean.reshape(ROWS, D)
    std264 = emb_std.reshape(ROWS, D)
    f0 = fc_small[:, 0].reshape(N_PAIRS, 2 * D)
    f1 = fc_small[:, 1].reshape(N_PAIRS, 2 * D)
    f2 = fc_small[:, 2].reshape(N_PAIRS, 2 * D)
    f3 = fc_small[:, 3].reshape(N_PAIRS, 2 * D)
    fa = fc_concat[:, :, :D].reshape(N_PAIRS, 2 * D)
    fb = fc_concat[:, :, D:].reshape(N_PAIRS, 2 * D)
    la = log_alpha.reshape(N_OPS)
    mir = jnp.asarray(_MIR)
    mjr = jnp.asarray(_MJR)
    r22 = jnp.asarray(_R22)
    pidx = jnp.asarray(_I_IDX.astype(np.int32))
    pjdx = jnp.asarray(_J_IDX.astype(np.int32))

    full2 = lambda t: (0, 0)
    out = pl.pallas_call(
        _nasp_kernel,
        grid=(B // TILE,),
        in_specs=[
            pl.BlockSpec(memory_space=pltpu.SMEM),
            pl.BlockSpec(memory_space=pltpu.SMEM),
            pl.BlockSpec(memory_space=pltpu.SMEM),
            pl.BlockSpec((TILE, N_COLS), lambda t: (t, 0)),
            pl.BlockSpec((ROWS, D), full2),
            pl.BlockSpec((ROWS, D), full2),
            pl.BlockSpec((TILE, D), lambda t: (t, 0)),
            pl.BlockSpec((N_PAIRS, 2 * D), full2),
            pl.BlockSpec((N_PAIRS, 2 * D), full2),
            pl.BlockSpec((N_PAIRS, 2 * D), full2),
            pl.BlockSpec((N_PAIRS, 2 * D), full2),
            pl.BlockSpec((N_PAIRS, 2 * D), full2),
            pl.BlockSpec((N_PAIRS, 2 * D), full2),
            pl.BlockSpec((ROWS, N_PAIRS), full2),
            pl.BlockSpec((ROWS, N_PAIRS), full2),
            pl.BlockSpec((N_COLS, ROWS), full2),
        ],
        out_specs=pl.BlockSpec((TILE, 2), lambda t: (t, 0)),
        out_shape=jax.ShapeDtypeStruct((B, 2), jnp.float32),
        scratch_shapes=[pltpu.VMEM((N_COLS, TILE, D), jnp.float32),
                        pltpu.VMEM((TILE, D), jnp.float32),
                        pltpu.VMEM((TILE, D), jnp.float32),
                        pltpu.VMEM((ROWS, D), jnp.float32),
                        pltpu.VMEM((ROWS, 2 * D), jnp.bfloat16),
                        pltpu.VMEM((ROWS, 2 * D), jnp.bfloat16),
                        pltpu.VMEM((ROWS, 2), jnp.bfloat16),
                        pltpu.VMEM((ROWS, 2), jnp.bfloat16)],
    )(la, pidx, pjdx, featT, mean264, std264, v, f0, f1, f2, f3, fa, fb,
      mir, mjr, r22)
    return out


# speedup vs baseline: 1.8367x; 1.8367x over previous
import jax
import jax.numpy as jnp
import numpy as np
from jax.experimental import pallas as pl
from jax.experimental.pallas import tpu as pltpu

N_COLS = 22
N_PAIRS = N_COLS * (N_COLS - 1) // 2
EMB_NUM = 12
ROWS = N_COLS * EMB_NUM
D = 64
N_OPS = 5
TILE = 1024

_I_IDX, _J_IDX = np.triu_indices(N_COLS, k=1)
_MIR = np.zeros((ROWS, N_PAIRS), np.float32)
_MJR = np.zeros((ROWS, N_PAIRS), np.float32)
for _p in range(N_PAIRS):
    _MIR[_I_IDX[_p] * EMB_NUM:(_I_IDX[_p] + 1) * EMB_NUM, _p] = 1.0
    _MJR[_J_IDX[_p] * EMB_NUM:(_J_IDX[_p] + 1) * EMB_NUM, _p] = 1.0

_HI = jax.lax.Precision.HIGHEST
_H3 = jax.lax.Precision.HIGH

_R22 = np.zeros((N_COLS, ROWS), np.float32)
for _c in range(N_COLS):
    _R22[_c, _c * EMB_NUM:(_c + 1) * EMB_NUM] = 1.0


def _nasp_kernel(la_ref, pi_ref, pj_ref, feat_ref, mean_ref, std_ref, v_ref,
                 f0_ref, f1_ref, f2_ref, f3_ref, fa_ref, fb_ref,
                 mir_ref, mjr_ref, r22_ref, out_ref, e_ref, acc0_ref, acc1_ref,
                 sp01_ref, sbh_ref, sbl_ref, pbh_ref, pbl_ref):
    logits = [la_ref[k] for k in range(N_OPS)]
    best = logits[0]
    sel = jnp.int32(0)
    for k in range(1, N_OPS):
        is_new = logits[k] > best
        best = jnp.where(is_new, logits[k], best)
        sel = jnp.where(is_new, jnp.int32(k), sel)
    w = [(sel == k).astype(jnp.float32) for k in range(N_OPS)]
    w0, w1, w2, w3, w4 = w
    hw2 = 0.5 * w2
    hw3 = 0.5 * w3

    @pl.when(pl.program_id(0) == 0)
    def _prep():
        sp01_ref[...] = jnp.log1p(jnp.exp(std_ref[...])) * 0.01
        s_w = w0 * f0_ref[...] + hw2 * f2_ref[...] + hw3 * f3_ref[...]
        gi = s_w + w4 * fa_ref[...]
        gj = s_w + w4 * fb_ref[...]
        wcolr = (jnp.dot(mir_ref[...], gi, preferred_element_type=jnp.float32,
                         precision=_HI) +
                 jnp.dot(mjr_ref[...], gj, preferred_element_type=jnp.float32,
                         precision=_HI))
        mean = mean_ref[...]
        p0 = jnp.sum(mean * wcolr[:, :D], axis=1, keepdims=True)
        p1 = jnp.sum(mean * wcolr[:, D:], axis=1, keepdims=True)
        pbig = jnp.concatenate([p0, p1], axis=1)
        sp01 = sp01_ref[...]
        sbig = jnp.concatenate([sp01, sp01], axis=1) * wcolr
        sbh = sbig.astype(jnp.bfloat16)
        sbh_ref[...] = sbh
        sbl_ref[...] = (sbig - sbh.astype(jnp.float32)).astype(jnp.bfloat16)
        pbh = pbig.astype(jnp.bfloat16)
        pbh_ref[...] = pbh
        pbl_ref[...] = (pbig - pbh.astype(jnp.float32)).astype(jnp.bfloat16)

    iota = jax.lax.broadcasted_iota(jnp.int32, (1, ROWS), 1)
    pattern = jax.lax.rem(iota, jnp.int32(EMB_NUM)).astype(jnp.float32)
    featr = jnp.dot(feat_ref[...].astype(jnp.float32), r22_ref[...],
                    preferred_element_type=jnp.float32)
    oh = (featr == pattern).astype(jnp.float32)

    ohb = oh.astype(jnp.bfloat16)
    lin = (jnp.dot(ohb, pbh_ref[...], preferred_element_type=jnp.float32) +
           jnp.dot(ohb, pbl_ref[...], preferred_element_type=jnp.float32))
    g = (jnp.dot(ohb, sbh_ref[...], preferred_element_type=jnp.float32) +
         jnp.dot(ohb, sbl_ref[...], preferred_element_type=jnp.float32))
    v = v_ref[...]
    h = g * jnp.concatenate([v, v], axis=1)
    lin0 = lin[:, 0:1] + jnp.sum(h[:, :D], axis=1, keepdims=True)
    lin1 = lin[:, 1:2] + jnp.sum(h[:, D:], axis=1, keepdims=True)
    out_ref[...] = jnp.concatenate([lin0, lin1], axis=1)

    @pl.when(w1 + w2 + w3 > 0.5)
    def _pairwise():
        acc0_ref[...] = jnp.zeros((TILE, D), jnp.float32)
        acc1_ref[...] = jnp.zeros((TILE, D), jnp.float32)
        for c in range(N_COLS):
            ohc = oh[:, c * EMB_NUM:(c + 1) * EMB_NUM]
            mu = jnp.dot(ohc, mean_ref[c * EMB_NUM:(c + 1) * EMB_NUM, :],
                         preferred_element_type=jnp.float32, precision=_HI)
            sp = jnp.dot(ohc, sp01_ref[c * EMB_NUM:(c + 1) * EMB_NUM, :],
                         preferred_element_type=jnp.float32, precision=_HI)
            e_ref[c] = mu + sp * v
        def body(p, carry):
            i = pi_ref[p]
            j = pj_ref[p]
            ei = e_ref[i]
            ej = e_ref[j]
            m = ei * ej
            a = jnp.abs(ei - ej)
            wm = w1 * f1_ref[pl.ds(p, 1), :]
            wa = hw2 * f2_ref[pl.ds(p, 1), :] - hw3 * f3_ref[pl.ds(p, 1), :]
            acc0_ref[...] += m * wm[:, :D] + a * wa[:, :D]
            acc1_ref[...] += m * wm[:, D:] + a * wa[:, D:]
            return carry

        jax.lax.fori_loop(0, N_PAIRS, body, 0)
        r0 = jnp.sum(acc0_ref[...], axis=1, keepdims=True)
        r1 = jnp.sum(acc1_ref[...], axis=1, keepdims=True)
        out_ref[...] += jnp.concatenate([r0, r1], axis=1)


def kernel(features, label, emb_mean, emb_std, fc_small, fc_concat, log_alpha, rand_array):
    del label
    B = features.shape[1]
    featT = features.T.astype(jnp.int32)
    v = rand_array[:B * D].reshape(B, D)
    mean264 = emb_mean.reshape(ROWS, D)
    std264 = emb_std.reshape(ROWS, D)
    f0 = fc_small[:, 0].reshape(N_PAIRS, 2 * D)
    f1 = fc_small[:, 1].reshape(N_PAIRS, 2 * D)
    f2 = fc_small[:, 2].reshape(N_PAIRS, 2 * D)
    f3 = fc_small[:, 3].reshape(N_PAIRS, 2 * D)
    fa = fc_concat[:, :, :D].reshape(N_PAIRS, 2 * D)
    fb = fc_concat[:, :, D:].reshape(N_PAIRS, 2 * D)
    la = log_alpha.reshape(N_OPS)
    mir = jnp.asarray(_MIR)
    mjr = jnp.asarray(_MJR)
    r22 = jnp.asarray(_R22)
    pidx = jnp.asarray(_I_IDX.astype(np.int32))
    pjdx = jnp.asarray(_J_IDX.astype(np.int32))

    full2 = lambda t: (0, 0)
    out = pl.pallas_call(
        _nasp_kernel,
        grid=(B // TILE,),
        in_specs=[
            pl.BlockSpec(memory_space=pltpu.SMEM),
            pl.BlockSpec(memory_space=pltpu.SMEM),
            pl.BlockSpec(memory_space=pltpu.SMEM),
            pl.BlockSpec((TILE, N_COLS), lambda t: (t, 0)),
            pl.BlockSpec((ROWS, D), full2),
            pl.BlockSpec((ROWS, D), full2),
            pl.BlockSpec((TILE, D), lambda t: (t, 0)),
            pl.BlockSpec((N_PAIRS, 2 * D), full2),
            pl.BlockSpec((N_PAIRS, 2 * D), full2),
            pl.BlockSpec((N_PAIRS, 2 * D), full2),
            pl.BlockSpec((N_PAIRS, 2 * D), full2),
            pl.BlockSpec((N_PAIRS, 2 * D), full2),
            pl.BlockSpec((N_PAIRS, 2 * D), full2),
            pl.BlockSpec((ROWS, N_PAIRS), full2),
            pl.BlockSpec((ROWS, N_PAIRS), full2),
            pl.BlockSpec((N_COLS, ROWS), full2),
        ],
        out_specs=pl.BlockSpec((TILE, 2), lambda t: (t, 0)),
        out_shape=jax.ShapeDtypeStruct((B, 2), jnp.float32),
        scratch_shapes=[pltpu.VMEM((N_COLS, TILE, D), jnp.float32),
                        pltpu.VMEM((TILE, D), jnp.float32),
                        pltpu.VMEM((TILE, D), jnp.float32),
                        pltpu.VMEM((ROWS, D), jnp.float32),
                        pltpu.VMEM((ROWS, 2 * D), jnp.bfloat16),
                        pltpu.VMEM((ROWS, 2 * D), jnp.bfloat16),
                        pltpu.VMEM((ROWS, 2), jnp.bfloat16),
                        pltpu.VMEM((ROWS, 2), jnp.bfloat16)],
    )(la, pidx, pjdx, featT, mean264, std264, v, f0, f1, f2, f3, fa, fb,
      mir, mjr, r22)
    return out


# speedup vs baseline: 1.9538x; 1.0637x over previous
import jax
import jax.numpy as jnp
import numpy as np
from jax.experimental import pallas as pl
from jax.experimental.pallas import tpu as pltpu

N_COLS = 22
N_PAIRS = N_COLS * (N_COLS - 1) // 2
EMB_NUM = 12
ROWS = N_COLS * EMB_NUM
D = 64
N_OPS = 5
TILE = 1024

_I_IDX, _J_IDX = np.triu_indices(N_COLS, k=1)
_MIR = np.zeros((ROWS, N_PAIRS), np.float32)
_MJR = np.zeros((ROWS, N_PAIRS), np.float32)
for _p in range(N_PAIRS):
    _MIR[_I_IDX[_p] * EMB_NUM:(_I_IDX[_p] + 1) * EMB_NUM, _p] = 1.0
    _MJR[_J_IDX[_p] * EMB_NUM:(_J_IDX[_p] + 1) * EMB_NUM, _p] = 1.0

_HI = jax.lax.Precision.HIGHEST
_H3 = jax.lax.Precision.HIGH

_R22 = np.zeros((N_COLS, ROWS), np.float32)
for _c in range(N_COLS):
    _R22[_c, _c * EMB_NUM:(_c + 1) * EMB_NUM] = 1.0


def _nasp_kernel(la_ref, pi_ref, pj_ref, feat_ref, mean_ref, std_ref, v_ref,
                 f0_ref, f1_ref, f2_ref, f3_ref, fa_ref, fb_ref,
                 mir_ref, mjr_ref, r22_ref, out_ref, e_ref, acc0_ref, acc1_ref,
                 sp01_ref, sbh_ref, sbl_ref, pbh_ref, pbl_ref):
    logits = [la_ref[k] for k in range(N_OPS)]
    best = logits[0]
    sel = jnp.int32(0)
    for k in range(1, N_OPS):
        is_new = logits[k] > best
        best = jnp.where(is_new, logits[k], best)
        sel = jnp.where(is_new, jnp.int32(k), sel)
    w = [(sel == k).astype(jnp.float32) for k in range(N_OPS)]
    w0, w1, w2, w3, w4 = w
    hw2 = 0.5 * w2
    hw3 = 0.5 * w3

    @pl.when(pl.program_id(0) == 0)
    def _prep():
        sp01_ref[...] = jnp.log1p(jnp.exp(std_ref[...])) * 0.01
        s_w = w0 * f0_ref[...] + hw2 * f2_ref[...] + hw3 * f3_ref[...]
        gi = s_w + w4 * fa_ref[...]
        gj = s_w + w4 * fb_ref[...]
        wcolr = (jnp.dot(mir_ref[...], gi, preferred_element_type=jnp.float32,
                         precision=_HI) +
                 jnp.dot(mjr_ref[...], gj, preferred_element_type=jnp.float32,
                         precision=_HI))
        mean = mean_ref[...]
        p0 = jnp.sum(mean * wcolr[:, :D], axis=1, keepdims=True)
        p1 = jnp.sum(mean * wcolr[:, D:], axis=1, keepdims=True)
        pbig = jnp.concatenate([p0, p1], axis=1)
        sp01 = sp01_ref[...]
        sbig = jnp.concatenate([sp01, sp01], axis=1) * wcolr
        sbh = sbig.astype(jnp.bfloat16)
        sbh_ref[...] = sbh
        sbl_ref[...] = (sbig - sbh.astype(jnp.float32)).astype(jnp.bfloat16)
        pbh = pbig.astype(jnp.bfloat16)
        pbh_ref[...] = pbh
        pbl_ref[...] = (pbig - pbh.astype(jnp.float32)).astype(jnp.bfloat16)

    iota = jax.lax.broadcasted_iota(jnp.int32, (1, ROWS), 1)
    pattern = jax.lax.rem(iota, jnp.int32(EMB_NUM)).astype(jnp.float32)
    featr = jax.lax.dot_general(feat_ref[...].astype(jnp.float32),
                                r22_ref[...], (((0,), (0,)), ((), ())),
                                preferred_element_type=jnp.float32)
    ohb = (featr == pattern).astype(jnp.bfloat16)

    lin = (jnp.dot(ohb, pbh_ref[...], preferred_element_type=jnp.float32) +
           jnp.dot(ohb, pbl_ref[...], preferred_element_type=jnp.float32))
    g = (jnp.dot(ohb, sbh_ref[...], preferred_element_type=jnp.float32) +
         jnp.dot(ohb, sbl_ref[...], preferred_element_type=jnp.float32))
    v = v_ref[...]
    h = g * jnp.concatenate([v, v], axis=1)
    lin0 = lin[:, 0:1] + jnp.sum(h[:, :D], axis=1, keepdims=True)
    lin1 = lin[:, 1:2] + jnp.sum(h[:, D:], axis=1, keepdims=True)
    out_ref[...] = jnp.concatenate([lin0, lin1], axis=1)

    @pl.when(w1 + w2 + w3 > 0.5)
    def _pairwise():
        acc0_ref[...] = jnp.zeros((TILE, D), jnp.float32)
        acc1_ref[...] = jnp.zeros((TILE, D), jnp.float32)
        oh = ohb.astype(jnp.float32)
        for c in range(N_COLS):
            ohc = oh[:, c * EMB_NUM:(c + 1) * EMB_NUM]
            mu = jnp.dot(ohc, mean_ref[c * EMB_NUM:(c + 1) * EMB_NUM, :],
                         preferred_element_type=jnp.float32, precision=_HI)
            sp = jnp.dot(ohc, sp01_ref[c * EMB_NUM:(c + 1) * EMB_NUM, :],
                         preferred_element_type=jnp.float32, precision=_HI)
            e_ref[c] = mu + sp * v
        def body(p, carry):
            i = pi_ref[p]
            j = pj_ref[p]
            ei = e_ref[i]
            ej = e_ref[j]
            m = ei * ej
            a = jnp.abs(ei - ej)
            wm = w1 * f1_ref[pl.ds(p, 1), :]
            wa = hw2 * f2_ref[pl.ds(p, 1), :] - hw3 * f3_ref[pl.ds(p, 1), :]
            acc0_ref[...] += m * wm[:, :D] + a * wa[:, :D]
            acc1_ref[...] += m * wm[:, D:] + a * wa[:, D:]
            return carry

        jax.lax.fori_loop(0, N_PAIRS, body, 0)
        r0 = jnp.sum(acc0_ref[...], axis=1, keepdims=True)
        r1 = jnp.sum(acc1_ref[...], axis=1, keepdims=True)
        out_ref[...] += jnp.concatenate([r0, r1], axis=1)


def kernel(features, label, emb_mean, emb_std, fc_small, fc_concat, log_alpha, rand_array):
    del label
    B = features.shape[1]
    feat = features.astype(jnp.int32)
    v = rand_array[:B * D].reshape(B, D)
    mean264 = emb_mean.reshape(ROWS, D)
    std264 = emb_std.reshape(ROWS, D)
    f0 = fc_small[:, 0].reshape(N_PAIRS, 2 * D)
    f1 = fc_small[:, 1].reshape(N_PAIRS, 2 * D)
    f2 = fc_small[:, 2].reshape(N_PAIRS, 2 * D)
    f3 = fc_small[:, 3].reshape(N_PAIRS, 2 * D)
    fa = fc_concat[:, :, :D].reshape(N_PAIRS, 2 * D)
    fb = fc_concat[:, :, D:].reshape(N_PAIRS, 2 * D)
    la = log_alpha.reshape(N_OPS)
    mir = jnp.asarray(_MIR)
    mjr = jnp.asarray(_MJR)
    r22 = jnp.asarray(_R22)
    pidx = jnp.asarray(_I_IDX.astype(np.int32))
    pjdx = jnp.asarray(_J_IDX.astype(np.int32))

    full2 = lambda t: (0, 0)
    out = pl.pallas_call(
        _nasp_kernel,
        grid=(B // TILE,),
        in_specs=[
            pl.BlockSpec(memory_space=pltpu.SMEM),
            pl.BlockSpec(memory_space=pltpu.SMEM),
            pl.BlockSpec(memory_space=pltpu.SMEM),
            pl.BlockSpec((N_COLS, TILE), lambda t: (0, t)),
            pl.BlockSpec((ROWS, D), full2),
            pl.BlockSpec((ROWS, D), full2),
            pl.BlockSpec((TILE, D), lambda t: (t, 0)),
            pl.BlockSpec((N_PAIRS, 2 * D), full2),
            pl.BlockSpec((N_PAIRS, 2 * D), full2),
            pl.BlockSpec((N_PAIRS, 2 * D), full2),
            pl.BlockSpec((N_PAIRS, 2 * D), full2),
            pl.BlockSpec((N_PAIRS, 2 * D), full2),
            pl.BlockSpec((N_PAIRS, 2 * D), full2),
            pl.BlockSpec((ROWS, N_PAIRS), full2),
            pl.BlockSpec((ROWS, N_PAIRS), full2),
            pl.BlockSpec((N_COLS, ROWS), full2),
        ],
        out_specs=pl.BlockSpec((TILE, 2), lambda t: (t, 0)),
        out_shape=jax.ShapeDtypeStruct((B, 2), jnp.float32),
        scratch_shapes=[pltpu.VMEM((N_COLS, TILE, D), jnp.float32),
                        pltpu.VMEM((TILE, D), jnp.float32),
                        pltpu.VMEM((TILE, D), jnp.float32),
                        pltpu.VMEM((ROWS, D), jnp.float32),
                        pltpu.VMEM((ROWS, 2 * D), jnp.bfloat16),
                        pltpu.VMEM((ROWS, 2 * D), jnp.bfloat16),
                        pltpu.VMEM((ROWS, 2), jnp.bfloat16),
                        pltpu.VMEM((ROWS, 2), jnp.bfloat16)],
    )(la, pidx, pjdx, feat, mean264, std264, v, f0, f1, f2, f3, fa, fb,
      mir, mjr, r22)
    return out


# speedup vs baseline: 1.9544x; 1.0003x over previous
import jax
import jax.numpy as jnp
import numpy as np
from jax.experimental import pallas as pl
from jax.experimental.pallas import tpu as pltpu

N_COLS = 22
N_PAIRS = N_COLS * (N_COLS - 1) // 2
EMB_NUM = 12
ROWS = N_COLS * EMB_NUM
D = 64
N_OPS = 5
TILE = 1024

_I_IDX, _J_IDX = np.triu_indices(N_COLS, k=1)
_MIR = np.zeros((ROWS, N_PAIRS), np.float32)
_MJR = np.zeros((ROWS, N_PAIRS), np.float32)
for _p in range(N_PAIRS):
    _MIR[_I_IDX[_p] * EMB_NUM:(_I_IDX[_p] + 1) * EMB_NUM, _p] = 1.0
    _MJR[_J_IDX[_p] * EMB_NUM:(_J_IDX[_p] + 1) * EMB_NUM, _p] = 1.0

_HI = jax.lax.Precision.HIGHEST

_R22 = np.zeros((N_COLS, ROWS), np.float32)
for _c in range(N_COLS):
    _R22[_c, _c * EMB_NUM:(_c + 1) * EMB_NUM] = 1.0


def _nasp_kernel(la_ref, pi_ref, pj_ref, feat_ref, mean_ref, std_ref, v_ref,
                 f0_ref, f1_ref, f2_ref, f3_ref, fa_ref, fb_ref,
                 mir_ref, mjr_ref, r22_ref, out_ref, e_ref, acc0_ref, acc1_ref,
                 sp01_ref, sbh_ref, sbl_ref, pbh_ref, pbl_ref):
    logits = [la_ref[k] for k in range(N_OPS)]
    best = logits[0]
    sel = jnp.int32(0)
    for k in range(1, N_OPS):
        is_new = logits[k] > best
        best = jnp.where(is_new, logits[k], best)
        sel = jnp.where(is_new, jnp.int32(k), sel)
    w = [(sel == k).astype(jnp.float32) for k in range(N_OPS)]
    w0, w1, w2, w3, w4 = w
    hw2 = 0.5 * w2
    hw3 = 0.5 * w3

    @pl.when(pl.program_id(0) == 0)
    def _prep():
        sp01_ref[...] = jnp.log1p(jnp.exp(std_ref[...])) * 0.01
        s_w = w0 * f0_ref[...] + hw2 * f2_ref[...] + hw3 * f3_ref[...]
        gi = s_w + w4 * fa_ref[...]
        gj = s_w + w4 * fb_ref[...]
        wcolr = (jnp.dot(mir_ref[...], gi, preferred_element_type=jnp.float32,
                         precision=_HI) +
                 jnp.dot(mjr_ref[...], gj, preferred_element_type=jnp.float32,
                         precision=_HI))
        mean = mean_ref[...]
        p0 = jnp.sum(mean * wcolr[:, :D], axis=1, keepdims=True)
        p1 = jnp.sum(mean * wcolr[:, D:], axis=1, keepdims=True)
        pbig = jnp.concatenate([p0, p1], axis=1)
        sp01 = sp01_ref[...]
        sbig = jnp.concatenate([sp01, sp01], axis=1) * wcolr
        sbh = sbig.astype(jnp.bfloat16)
        sbh_ref[...] = sbh
        sbl_ref[...] = (sbig - sbh.astype(jnp.float32)).astype(jnp.bfloat16)
        pbh = pbig.astype(jnp.bfloat16)
        pbh_ref[...] = pbh
        pbl_ref[...] = (pbig - pbh.astype(jnp.float32)).astype(jnp.bfloat16)

    iota = jax.lax.broadcasted_iota(jnp.int32, (1, ROWS), 1)
    pattern = jax.lax.rem(iota, jnp.int32(EMB_NUM)).astype(jnp.float32)
    featr = jax.lax.dot_general(feat_ref[...].astype(jnp.float32),
                                r22_ref[...], (((0,), (0,)), ((), ())),
                                preferred_element_type=jnp.float32)
    ohb = (featr == pattern).astype(jnp.bfloat16)

    lin = (jnp.dot(ohb, pbh_ref[...], preferred_element_type=jnp.float32) +
           jnp.dot(ohb, pbl_ref[...], preferred_element_type=jnp.float32))
    g = (jnp.dot(ohb, sbh_ref[...], preferred_element_type=jnp.float32) +
         jnp.dot(ohb, sbl_ref[...], preferred_element_type=jnp.float32))
    v = v_ref[...]
    h = g * jnp.concatenate([v, v], axis=1)
    lin0 = lin[:, 0:1] + jnp.sum(h[:, :D], axis=1, keepdims=True)
    lin1 = lin[:, 1:2] + jnp.sum(h[:, D:], axis=1, keepdims=True)
    out_ref[...] = jnp.concatenate([lin0, lin1], axis=1)

    @pl.when(w1 + w2 + w3 > 0.5)
    def _pairwise():
        acc0_ref[...] = jnp.zeros((TILE, D), jnp.float32)
        acc1_ref[...] = jnp.zeros((TILE, D), jnp.float32)
        oh = ohb.astype(jnp.float32)
        for c in range(N_COLS):
            ohc = oh[:, c * EMB_NUM:(c + 1) * EMB_NUM]
            mu = jnp.dot(ohc, mean_ref[c * EMB_NUM:(c + 1) * EMB_NUM, :],
                         preferred_element_type=jnp.float32, precision=_HI)
            sp = jnp.dot(ohc, sp01_ref[c * EMB_NUM:(c + 1) * EMB_NUM, :],
                         preferred_element_type=jnp.float32, precision=_HI)
            e_ref[c] = mu + sp * v
        def body(p, carry):
            i = pi_ref[p]
            j = pj_ref[p]
            ei = e_ref[i]
            ej = e_ref[j]
            m = ei * ej
            a = jnp.abs(ei - ej)
            wm = w1 * f1_ref[pl.ds(p, 1), :]
            wa = hw2 * f2_ref[pl.ds(p, 1), :] - hw3 * f3_ref[pl.ds(p, 1), :]
            acc0_ref[...] += m * wm[:, :D] + a * wa[:, :D]
            acc1_ref[...] += m * wm[:, D:] + a * wa[:, D:]
            return carry

        jax.lax.fori_loop(0, N_PAIRS, body, 0)
        r0 = jnp.sum(acc0_ref[...], axis=1, keepdims=True)
        r1 = jnp.sum(acc1_ref[...], axis=1, keepdims=True)
        out_ref[...] += jnp.concatenate([r0, r1], axis=1)


def kernel(features, label, emb_mean, emb_std, fc_small, fc_concat, log_alpha, rand_array):
    del label
    B = features.shape[1]
    feat = features.astype(jnp.int32)
    v = rand_array[:B * D].reshape(B, D)
    mean264 = emb_mean.reshape(ROWS, D)
    std264 = emb_std.reshape(ROWS, D)
    f0 = fc_small[:, 0].reshape(N_PAIRS, 2 * D)
    f1 = fc_small[:, 1].reshape(N_PAIRS, 2 * D)
    f2 = fc_small[:, 2].reshape(N_PAIRS, 2 * D)
    f3 = fc_small[:, 3].reshape(N_PAIRS, 2 * D)
    fa = fc_concat[:, :, :D].reshape(N_PAIRS, 2 * D)
    fb = fc_concat[:, :, D:].reshape(N_PAIRS, 2 * D)
    la = log_alpha.reshape(N_OPS)
    mir = jnp.asarray(_MIR)
    mjr = jnp.asarray(_MJR)
    r22 = jnp.asarray(_R22)
    pidx = jnp.asarray(_I_IDX.astype(np.int32))
    pjdx = jnp.asarray(_J_IDX.astype(np.int32))

    full2 = lambda t: (0, 0)
    out = pl.pallas_call(
        _nasp_kernel,
        grid=(B // TILE,),
        in_specs=[
            pl.BlockSpec(memory_space=pltpu.SMEM),
            pl.BlockSpec(memory_space=pltpu.SMEM),
            pl.BlockSpec(memory_space=pltpu.SMEM),
            pl.BlockSpec((N_COLS, TILE), lambda t: (0, t)),
            pl.BlockSpec((ROWS, D), full2),
            pl.BlockSpec((ROWS, D), full2),
            pl.BlockSpec((TILE, D), lambda t: (t, 0)),
            pl.BlockSpec((N_PAIRS, 2 * D), full2),
            pl.BlockSpec((N_PAIRS, 2 * D), full2),
            pl.BlockSpec((N_PAIRS, 2 * D), full2),
            pl.BlockSpec((N_PAIRS, 2 * D), full2),
            pl.BlockSpec((N_PAIRS, 2 * D), full2),
            pl.BlockSpec((N_PAIRS, 2 * D), full2),
            pl.BlockSpec((ROWS, N_PAIRS), full2),
            pl.BlockSpec((ROWS, N_PAIRS), full2),
            pl.BlockSpec((N_COLS, ROWS), full2),
        ],
        out_specs=pl.BlockSpec((TILE, 2), lambda t: (t, 0)),
        out_shape=jax.ShapeDtypeStruct((B, 2), jnp.float32),
        scratch_shapes=[pltpu.VMEM((N_COLS, TILE, D), jnp.float32),
                        pltpu.VMEM((TILE, D), jnp.float32),
                        pltpu.VMEM((TILE, D), jnp.float32),
                        pltpu.VMEM((ROWS, D), jnp.float32),
                        pltpu.VMEM((ROWS, 2 * D), jnp.bfloat16),
                        pltpu.VMEM((ROWS, 2 * D), jnp.bfloat16),
                        pltpu.VMEM((ROWS, 2), jnp.bfloat16),
                        pltpu.VMEM((ROWS, 2), jnp.bfloat16)],
    )(la, pidx, pjdx, feat, mean264, std264, v, f0, f1, f2, f3, fa, fb,
      mir, mjr, r22)
    return out
